# baseline (speedup 1.0000x reference)
import functools

import jax
import jax.numpy as jnp
from jax import lax
from jax.experimental import pallas as pl
from jax.experimental.pallas import tpu as pltpu
from jax.experimental.pallas import tpu_sc as plsc

VOCAB = 49408
EMBED = 1024
MAX_POS = 77
BATCH = 1024
SEQ = 77

B = BATCH * SEQ
NW = 32
B_PER_W = B // NW
CHUNK = 16
NCHUNK = B_PER_W // CHUNK
LANES = 16


def _body(ids_hbm, tok_hbm, pos_hbm, out_hbm, idx_v, rows_v, pos_v, sem):
    wid = lax.axis_index("s") * 2 + lax.axis_index("c")
    base = wid * B_PER_W
    pltpu.sync_copy(pos_hbm, pos_v)

    def chunk_body(c, carry):
        row0 = base + c * CHUNK
        pltpu.sync_copy(ids_hbm.at[pl.ds(row0, CHUNK)], idx_v)
        pltpu.async_copy(tok_hbm.at[idx_v], rows_v, sem).wait()

        def row_body(r, carry2):
            p = lax.rem(c * CHUNK + r, MAX_POS)
            for j in range(EMBED // LANES):
                sl = pl.ds(j * LANES, LANES)
                rows_v[r, sl] = rows_v[r, sl] + pos_v[p, sl]
            return carry2

        lax.fori_loop(0, CHUNK, row_body, 0)
        pltpu.sync_copy(rows_v, out_hbm.at[pl.ds(row0, CHUNK)])
        return carry

    lax.fori_loop(0, NCHUNK, chunk_body, 0)


@jax.jit
def kernel(input_ids, token_table, position_table):
    ids = input_ids.reshape(-1).astype(jnp.int32)
    mesh = plsc.VectorSubcoreMesh(core_axis_name="c", subcore_axis_name="s")
    out = pl.kernel(
        _body,
        mesh=mesh,
        out_type=jax.ShapeDtypeStruct((B, EMBED), jnp.float32),
        scratch_types=[
            pltpu.VMEM((CHUNK,), jnp.int32),
            pltpu.VMEM((CHUNK, EMBED), jnp.float32),
            pltpu.VMEM((MAX_POS, EMBED), jnp.float32),
            pltpu.SemaphoreType.DMA,
        ],
    )(ids, token_table, position_table)
    return out.reshape(BATCH, SEQ, EMBED)


# speedup vs baseline: 1.4185x; 1.4185x over previous
import functools

import jax
import jax.numpy as jnp
from jax import lax
from jax.experimental import pallas as pl
from jax.experimental.pallas import tpu as pltpu
from jax.experimental.pallas import tpu_sc as plsc

VOCAB = 49408
EMBED = 1024
MAX_POS = 77
BATCH = 1024
SEQ = 77

NQ = 4
QD = EMBED // NQ
NG = 8
B_PER_G = BATCH // NG
ROWS = 80
LANES = 16


def _add_pos(rows_v, pos_v):
    def row_body(r, carry):
        for j in range(QD // LANES):
            sl = pl.ds(j * LANES, LANES)
            rows_v[r, sl] = rows_v[r, sl] + pos_v[r, sl]
        return carry

    lax.fori_loop(0, MAX_POS, row_body, 0)


def _body(ids_hbm, tok_hbm, pos_hbm, out_hbm,
          idx0, idx1, qidx0, qidx1, rows0, rows1, pos_v,
          gsem0, gsem1, ssem0, ssem1):
    wid = lax.axis_index("s") * 2 + lax.axis_index("c")
    g = wid // NQ
    h = wid % NQ
    b0 = g * B_PER_G
    col = h * QD

    idxs = (idx0, idx1)
    qidxs = (qidx0, qidx1)
    rows = (rows0, rows1)
    gsems = (gsem0, gsem1)
    ssems = (ssem0, ssem1)

    pltpu.sync_copy(pos_hbm.at[:, pl.ds(col, QD)], pos_v)

    hvec = jnp.full((LANES,), h, jnp.int32)

    def stage_and_gather(c, p):
        pltpu.sync_copy(ids_hbm.at[b0 + c], idxs[p])
        for j in range(ROWS // LANES):
            sl = pl.ds(j * LANES, LANES)
            qidxs[p][sl] = idxs[p][sl] * NQ + hvec
        pltpu.async_copy(tok_hbm.at[qidxs[p]], rows[p], gsems[p])

    def wait_gather(p):
        pltpu.make_async_copy(tok_hbm.at[qidxs[p]], rows[p], gsems[p]).wait()

    def start_store(c, p):
        pltpu.async_copy(rows[p].at[pl.ds(0, MAX_POS)],
                         out_hbm.at[b0 + c, :, pl.ds(col, QD)], ssems[p])

    def wait_store(c, p):
        pltpu.make_async_copy(rows[p].at[pl.ds(0, MAX_POS)],
                              out_hbm.at[b0 + c, :, pl.ds(col, QD)],
                              ssems[p]).wait()

    stage_and_gather(0, 0)

    def pair_body(t, carry):
        for p in (0, 1):
            c = 2 * t + p
            if p == 0:
                @pl.when(t > 0)
                def _():
                    wait_store(c - 1, 1)
            else:
                wait_store(c - 1, 0)
            if p == 0:
                stage_and_gather(c + 1, 1)
            else:
                @pl.when(t < B_PER_G // 2 - 1)
                def _():
                    stage_and_gather(c + 1, 0)
            wait_gather(p)
            _add_pos(rows[p], pos_v)
            start_store(c, p)
        return carry

    lax.fori_loop(0, B_PER_G // 2, pair_body, 0)
    wait_store(B_PER_G - 1, 1)


@jax.jit
def kernel(input_ids, token_table, position_table):
    ids = jnp.pad(input_ids.astype(jnp.int32), ((0, 0), (0, ROWS - SEQ)))
    tok4 = token_table.reshape(VOCAB * NQ, QD)
    mesh = plsc.VectorSubcoreMesh(core_axis_name="c", subcore_axis_name="s")
    return pl.kernel(
        _body,
        mesh=mesh,
        out_type=jax.ShapeDtypeStruct((BATCH, SEQ, EMBED), jnp.float32),
        compiler_params=pltpu.CompilerParams(use_tc_tiling_on_sc=False),
        scratch_types=[
            pltpu.VMEM((ROWS,), jnp.int32),
            pltpu.VMEM((ROWS,), jnp.int32),
            pltpu.VMEM((ROWS,), jnp.int32),
            pltpu.VMEM((ROWS,), jnp.int32),
            pltpu.VMEM((ROWS, QD), jnp.float32),
            pltpu.VMEM((ROWS, QD), jnp.float32),
            pltpu.VMEM((MAX_POS, QD), jnp.float32),
            pltpu.SemaphoreType.DMA,
            pltpu.SemaphoreType.DMA,
            pltpu.SemaphoreType.DMA,
            pltpu.SemaphoreType.DMA,
        ],
    )(ids, tok4, position_table)


# speedup vs baseline: 2.2613x; 1.5942x over previous
import functools

import jax
import jax.numpy as jnp
from jax import lax
from jax.experimental import pallas as pl
from jax.experimental.pallas import tpu as pltpu
from jax.experimental.pallas import tpu_sc as plsc

VOCAB = 49408
EMBED = 1024
MAX_POS = 77
BATCH = 1024
SEQ = 77

NQ = 4
QD = EMBED // NQ
NG = 8
B_PER_G = BATCH // NG
RP = 80
LANES = 16


def _add_pos(rows_v, pos_v):
    def row_body(r, carry):
        for j in range(QD // LANES):
            sl = pl.ds(j * LANES, LANES)
            rows_v[r, sl] = rows_v[r, sl] + pos_v[r, sl]
        return carry

    lax.fori_loop(0, MAX_POS, row_body, 0)


def _body(ids_hbm, tok_hbm, pos_hbm, out_hbm,
          idx0, idx1, rows0, rows1, pos_v,
          gsem0, gsem1, ssem0, ssem1):
    wid = lax.axis_index("s") * 2 + lax.axis_index("c")
    g = wid // NQ
    h = wid % NQ
    b0 = g * B_PER_G
    col = h * QD

    idxs = (idx0, idx1)
    rows = (rows0, rows1)
    gsems = (gsem0, gsem1)
    ssems = (ssem0, ssem1)

    pltpu.sync_copy(pos_hbm.at[:, pl.ds(col, QD)], pos_v)

    def stage_and_gather(c, p):
        pltpu.sync_copy(ids_hbm.at[pl.ds((b0 + c) * RP, RP)], idxs[p])
        pltpu.async_copy(tok_hbm.at[idxs[p], pl.ds(col, QD)], rows[p],
                         gsems[p])

    def wait_gather(p):
        pltpu.make_async_copy(tok_hbm.at[idxs[p], pl.ds(col, QD)], rows[p],
                              gsems[p]).wait()

    def start_store(c, p):
        pltpu.async_copy(rows[p],
                         out_hbm.at[pl.ds((b0 + c) * RP, RP), pl.ds(col, QD)],
                         ssems[p])

    def wait_store(c, p):
        pltpu.make_async_copy(
            rows[p],
            out_hbm.at[pl.ds((b0 + c) * RP, RP), pl.ds(col, QD)],
            ssems[p]).wait()

    stage_and_gather(0, 0)

    def pair_body(t, carry):
        for p in (0, 1):
            c = 2 * t + p
            if p == 0:
                @pl.when(t > 0)
                def _():
                    wait_store(c - 1, 1)
            else:
                wait_store(c - 1, 0)
            if p == 0:
                stage_and_gather(c + 1, 1)
            else:
                @pl.when(t < B_PER_G // 2 - 1)
                def _():
                    stage_and_gather(c + 1, 0)
            wait_gather(p)
            _add_pos(rows[p], pos_v)
            start_store(c, p)
        return carry

    lax.fori_loop(0, B_PER_G // 2, pair_body, 0)
    wait_store(B_PER_G - 1, 1)


@jax.jit
def kernel(input_ids, token_table, position_table):
    ids = jnp.pad(input_ids.astype(jnp.int32), ((0, 0), (0, RP - SEQ)))
    ids = ids.reshape(-1)
    mesh = plsc.VectorSubcoreMesh(core_axis_name="c", subcore_axis_name="s")
    out = pl.kernel(
        _body,
        mesh=mesh,
        out_type=jax.ShapeDtypeStruct((BATCH * RP, EMBED), jnp.float32),
        scratch_types=[
            pltpu.VMEM((RP,), jnp.int32),
            pltpu.VMEM((RP,), jnp.int32),
            pltpu.VMEM((RP, QD), jnp.float32),
            pltpu.VMEM((RP, QD), jnp.float32),
            pltpu.VMEM((MAX_POS, QD), jnp.float32),
            pltpu.SemaphoreType.DMA,
            pltpu.SemaphoreType.DMA,
            pltpu.SemaphoreType.DMA,
            pltpu.SemaphoreType.DMA,
        ],
    )(ids, token_table, position_table)
    return out.reshape(BATCH, RP, EMBED)[:, :SEQ, :]


# speedup vs baseline: 2.3313x; 1.0309x over previous
import functools

import jax
import jax.numpy as jnp
from jax import lax
from jax.experimental import pallas as pl
from jax.experimental.pallas import tpu as pltpu
from jax.experimental.pallas import tpu_sc as plsc

VOCAB = 49408
EMBED = 1024
MAX_POS = 77
BATCH = 1024
SEQ = 77

NQ = 4
QD = EMBED // NQ
NG = 8
B_PER_G = BATCH // NG
RP = 80
LANES = 16


def _add_pos(rows_v, pos_v, out_v):
    def row_body(r, carry):
        for j in range(QD // LANES):
            sl = pl.ds(j * LANES, LANES)
            out_v[r, sl] = rows_v[r, sl] + pos_v[r, sl]
        return carry

    lax.fori_loop(0, MAX_POS, row_body, 0)


def _body(ids_hbm, tok_hbm, pos_hbm, out_hbm,
          idx0, idx1, rows0, rows1, st0, st1, pos_v,
          gsem0, gsem1, ssem0, ssem1):
    wid = lax.axis_index("s") * 2 + lax.axis_index("c")
    g = wid // NQ
    h = wid % NQ
    b0 = g * B_PER_G
    col = h * QD

    idxs = (idx0, idx1)
    rows = (rows0, rows1)
    sts = (st0, st1)
    gsems = (gsem0, gsem1)
    ssems = (ssem0, ssem1)

    pltpu.sync_copy(pos_hbm.at[:, pl.ds(col, QD)], pos_v)

    def stage_and_gather(c, p):
        pltpu.sync_copy(ids_hbm.at[pl.ds((b0 + c) * RP, RP)], idxs[p])
        pltpu.async_copy(tok_hbm.at[idxs[p], pl.ds(col, QD)], rows[p],
                         gsems[p])

    def wait_gather(p):
        pltpu.make_async_copy(tok_hbm.at[idxs[p], pl.ds(col, QD)], rows[p],
                              gsems[p]).wait()

    def start_store(c, p):
        pltpu.async_copy(sts[p], out_hbm.at[b0 + c, :, pl.ds(col, QD)],
                         ssems[p])

    def wait_store(c, p):
        pltpu.make_async_copy(sts[p], out_hbm.at[b0 + c, :, pl.ds(col, QD)],
                              ssems[p]).wait()

    stage_and_gather(0, 0)

    def pair_body(t, carry):
        for p in (0, 1):
            c = 2 * t + p
            if p == 0:
                stage_and_gather(c + 1, 1)
            else:
                @pl.when(t < B_PER_G // 2 - 1)
                def _():
                    stage_and_gather(c + 1, 0)
            wait_gather(p)
            @pl.when(t > 0)
            def _():
                wait_store(c - 2, p)
            _add_pos(rows[p], pos_v, sts[p])
            start_store(c, p)
        return carry

    lax.fori_loop(0, B_PER_G // 2, pair_body, 0)
    wait_store(B_PER_G - 2, 0)
    wait_store(B_PER_G - 1, 1)


@jax.jit
def kernel(input_ids, token_table, position_table):
    ids = jnp.pad(input_ids.astype(jnp.int32), ((0, 0), (0, RP - SEQ)))
    ids = ids.reshape(-1)
    mesh = plsc.VectorSubcoreMesh(core_axis_name="c", subcore_axis_name="s")
    return pl.kernel(
        _body,
        mesh=mesh,
        out_type=jax.ShapeDtypeStruct((BATCH, SEQ, EMBED), jnp.float32),
        scratch_types=[
            pltpu.VMEM((RP,), jnp.int32),
            pltpu.VMEM((RP,), jnp.int32),
            pltpu.VMEM((RP, QD), jnp.float32),
            pltpu.VMEM((RP, QD), jnp.float32),
            pltpu.VMEM((MAX_POS, QD), jnp.float32),
            pltpu.VMEM((MAX_POS, QD), jnp.float32),
            pltpu.VMEM((MAX_POS, QD), jnp.float32),
            pltpu.SemaphoreType.DMA,
            pltpu.SemaphoreType.DMA,
            pltpu.SemaphoreType.DMA,
            pltpu.SemaphoreType.DMA,
        ],
    )(ids, token_table, position_table)
